# baseline (speedup 1.0000x reference)
import functools

import jax
import jax.numpy as jnp
from jax.experimental import pallas as pl
from jax.experimental.pallas import tpu as pltpu


def _masked(W, k):
    bits = jax.lax.bitcast_convert_type(jnp.abs(W), jnp.int32)

    def body(_, lo_hi):
        lo, hi = lo_hi
        mid = lo + (hi - lo + 1) // 2
        cnt = jnp.sum((bits >= mid).astype(jnp.int32))
        ge = cnt >= k
        return jnp.where(ge, mid, lo), jnp.where(ge, hi, mid - 1)

    lo, _ = jax.lax.fori_loop(
        0, 31, body, (jnp.int32(0), jnp.int32(0x7F7FFFFF))
    )
    return jnp.where(bits >= lo, W, jnp.zeros_like(W))


def _fused_kernel(
    k1, k2, k3,
    x_ref, w1_ref, b1_ref, w2_ref, b2_ref, w3_ref, b3_ref,
    o_ref, w1s, w2s, w3s,
):
    @pl.when(pl.program_id(0) == 0)
    def _():
        w1s[...] = _masked(w1_ref[...], k1)
        w2s[...] = _masked(w2_ref[...], k2)
        w3s[...] = _masked(w3_ref[...], k3)

    dn = (((1,), (1,)), ((), ()))
    h = jax.lax.dot_general(
        x_ref[...], w1s[...], dn, preferred_element_type=jnp.float32
    ) + b1_ref[...]
    h = jnp.maximum(h, 0.0)
    h = jax.lax.dot_general(
        h, w2s[...], dn, preferred_element_type=jnp.float32
    ) + b2_ref[...]
    h = jnp.maximum(h, 0.0)
    o_ref[...] = jnp.sum(h * w3s[...], axis=1, keepdims=True) + b3_ref[0, 0]


def _k_of(numel, p_forward):
    return max(1, int(round((1.0 - p_forward) * numel)))


def kernel(X, W_in, b_in, W_h1, b_h1, W_out, b_out):
    B, d_in = X.shape
    d_h = W_in.shape[0]
    d_out = W_out.shape[0]

    k1 = _k_of(W_in.size, 0.6)
    k2 = _k_of(W_h1.size, 0.7)
    k3 = _k_of(W_out.size, 0.6)

    tile = 2048 if B % 2048 == 0 else B
    grid = (B // tile,)

    out = pl.pallas_call(
        functools.partial(_fused_kernel, k1, k2, k3),
        grid=grid,
        in_specs=[
            pl.BlockSpec((tile, d_in), lambda i: (i, 0)),
            pl.BlockSpec((d_h, d_in), lambda i: (0, 0)),
            pl.BlockSpec((1, d_h), lambda i: (0, 0)),
            pl.BlockSpec((d_h, d_h), lambda i: (0, 0)),
            pl.BlockSpec((1, d_h), lambda i: (0, 0)),
            pl.BlockSpec((d_out, d_h), lambda i: (0, 0)),
            pl.BlockSpec((1, d_out), lambda i: (0, 0)),
        ],
        out_specs=pl.BlockSpec((tile, d_out), lambda i: (i, 0)),
        out_shape=jax.ShapeDtypeStruct((B, d_out), X.dtype),
        scratch_shapes=[
            pltpu.VMEM((d_h, d_in), jnp.float32),
            pltpu.VMEM((d_h, d_h), jnp.float32),
            pltpu.VMEM((d_out, d_h), jnp.float32),
        ],
    )(
        X,
        W_in,
        b_in.reshape(1, d_h),
        W_h1,
        b_h1.reshape(1, d_h),
        W_out,
        b_out.reshape(1, d_out),
    )
    return out
